# baseline (speedup 1.0000x reference)
import functools

import jax
import jax.numpy as jnp
from jax.experimental import pallas as pl
from jax.experimental.pallas import tpu as pltpu


def _add_pe_kernel(x_ref, t_ref, o_ref, pe_ref, *, H):
    @pl.when(pl.program_id(0) == 0)
    def _():
        pe_ref[...] = t_ref[:H, :].T

    o_ref[...] = x_ref[...] + pe_ref[...][None, :, :]


def kernel(x, pos, pos_table):
    del pos
    B, C, H = x.shape
    NP, _ = pos_table.shape
    BB = 8

    return pl.pallas_call(
        functools.partial(_add_pe_kernel, H=H),
        grid=(B // BB,),
        in_specs=[
            pl.BlockSpec((BB, C, H), lambda i: (i, 0, 0)),
            pl.BlockSpec((NP, C), lambda i: (0, 0)),
        ],
        out_specs=pl.BlockSpec((BB, C, H), lambda i: (i, 0, 0)),
        out_shape=jax.ShapeDtypeStruct((B, C, H), x.dtype),
        scratch_shapes=[pltpu.VMEM((C, H), jnp.float32)],
    )(x, pos_table)


# speedup vs baseline: 1.1160x; 1.1160x over previous
import functools

import jax
import jax.numpy as jnp
from jax.experimental import pallas as pl
from jax.experimental.pallas import tpu as pltpu


def _add_pe_kernel(x_ref, t_ref, o_ref, pe_ref, *, H):
    @pl.when(pl.program_id(0) == 0)
    def _():
        pe_ref[...] = t_ref[:H, :].T

    o_ref[...] = x_ref[...] + pe_ref[...][None, :, :]


def kernel(x, pos, pos_table):
    del pos
    B, C, H = x.shape
    NP, _ = pos_table.shape
    BB = 32

    return pl.pallas_call(
        functools.partial(_add_pe_kernel, H=H),
        grid=(B // BB,),
        in_specs=[
            pl.BlockSpec((BB, C, H), lambda i: (i, 0, 0)),
            pl.BlockSpec((NP, C), lambda i: (0, 0)),
        ],
        out_specs=pl.BlockSpec((BB, C, H), lambda i: (i, 0, 0)),
        out_shape=jax.ShapeDtypeStruct((B, C, H), x.dtype),
        scratch_shapes=[pltpu.VMEM((C, H), jnp.float32)],
    )(x, pos_table)
